# baseline (speedup 1.0000x reference)
import functools

import jax
import jax.numpy as jnp
from jax import lax
from jax.experimental import pallas as pl
from jax.experimental.pallas import tpu as pltpu
from jax.experimental.pallas import tpu_sc as plsc

_LANES = 16


@functools.cache
def _build(B, S, N, D):
    info = plsc.get_sparse_core_info()
    nw = info.num_cores * info.num_subcores
    cols = S // nw
    nsb = cols // _LANES

    mesh = plsc.VectorSubcoreMesh(core_axis_name="c", subcore_axis_name="s")

    @functools.partial(
        pl.kernel,
        mesh=mesh,
        out_type=jax.ShapeDtypeStruct((B, D, S), jnp.float32),
        scratch_types=[
            pltpu.VMEM((N, D), jnp.float32),
            pltpu.SMEM((N * D,), jnp.float32),
            pltpu.VMEM((D, cols), jnp.float32),
            pltpu.VMEM((N, cols), jnp.float32),
            pltpu.VMEM((N, cols), jnp.float32),
            pltpu.VMEM((D, cols), jnp.float32),
            pltpu.VMEM((D, cols), jnp.float32),
            pltpu.SemaphoreType.DMA,
            pltpu.SemaphoreType.DMA,
            pltpu.SemaphoreType.DMA,
            pltpu.SemaphoreType.DMA,
        ],
    )
    def sc_kernel(xt_hbm, a_hbm, pt_hbm, out_hbm,
                  a_v, a_sm, p_v, x0_v, x1_v, o0_v, o1_v,
                  sx0, sx1, so0, so1):
        cid = lax.axis_index("c")
        sid = lax.axis_index("s")
        wid = sid * info.num_cores + cid
        s0 = wid * cols

        pltpu.sync_copy(a_hbm, a_v)
        pltpu.sync_copy(pt_hbm.at[:, pl.ds(s0, cols)], p_v)

        def spill_body(jv, carry):
            row = jv // (D // _LANES)
            col = (jv % (D // _LANES)) * _LANES
            vec = a_v[row, pl.ds(col, _LANES)]
            for k in range(_LANES):
                a_sm[jv * _LANES + k] = vec[k]
            return carry

        lax.fori_loop(0, (N * D) // _LANES, spill_body, 0)

        tgrp = 4

        def compute(x_v, o_v):
            def sbc_body(sbc, carry2):
                xs = [[x_v[n, pl.ds((sbc * tgrp + t) * _LANES, _LANES)]
                       for t in range(tgrp)] for n in range(N)]

                def d_body(d, carry3):
                    an = [a_sm[n * D + d] for n in range(N)]
                    for t in range(tgrp):
                        sl = pl.ds((sbc * tgrp + t) * _LANES, _LANES)
                        acc = p_v[d, sl]
                        for n in range(N):
                            acc = acc + an[n] * xs[n][t]
                        o_v[d, sl] = acc
                    return carry3

                lax.fori_loop(0, D, d_body, 0)
                return carry2

            lax.fori_loop(0, nsb // tgrp, sbc_body, 0)

        def fetch_x(b, x_v, sem):
            bc = jnp.minimum(b, B - 1)
            pltpu.async_copy(xt_hbm.at[bc, :, pl.ds(s0, cols)], x_v, sem)

        def wait_x(x_v, sem):
            pltpu.make_async_copy(
                xt_hbm.at[0, :, pl.ds(s0, cols)], x_v, sem).wait()

        def wait_out(o_v, sem):
            pltpu.make_async_copy(
                o_v, out_hbm.at[0, :, pl.ds(s0, cols)], sem).wait()

        fetch_x(0, x0_v, sx0)

        def batch_pair(g, carry):
            b0 = 2 * g
            fetch_x(b0 + 1, x1_v, sx1)
            wait_x(x0_v, sx0)

            @pl.when(g > 0)
            def _():
                wait_out(o0_v, so0)

            compute(x0_v, o0_v)
            pltpu.async_copy(o0_v, out_hbm.at[b0, :, pl.ds(s0, cols)], so0)

            fetch_x(b0 + 2, x0_v, sx0)
            wait_x(x1_v, sx1)

            @pl.when(g > 0)
            def _():
                wait_out(o1_v, so1)

            compute(x1_v, o1_v)
            pltpu.async_copy(o1_v, out_hbm.at[b0 + 1, :, pl.ds(s0, cols)], so1)
            return carry

        lax.fori_loop(0, B // 2, batch_pair, 0)

        wait_x(x0_v, sx0)
        wait_out(o0_v, so0)
        wait_out(o1_v, so1)

    return sc_kernel


def kernel(x, allele_embedding, position_table):
    B, S, N = x.shape
    D = allele_embedding.shape[1]
    xt = x.transpose(0, 2, 1)
    pt = position_table.T
    out_t = _build(B, S, N, D)(xt, allele_embedding, pt)
    return out_t.transpose(0, 2, 1)
